# baseline (speedup 1.0000x reference)
import functools

import jax
import jax.numpy as jnp
from jax import lax
from jax.experimental import pallas as pl
from jax.experimental.pallas import tpu as pltpu
from jax.experimental.pallas import tpu_sc as plsc

B, T, L, D, H, V = 8, 16, 512, 1024, 16, 100000
R = B * T
NC, NS = 2, 16
G8 = 8
NG = R // G8
WMAX = 6144
NFULL = 16
W_TAIL = V - NFULL * WMAX
TAIL_V0 = NFULL * WMAX
H0_WINS = list(range(8))
H1_WINS = list(range(8, 16))

_LN2 = 0.6931471805599453

def _stats_body(x_ref, dec_ref, w_ref, b_ref, cc_ref):
    x = x_ref[0]
    m = jnp.max(x, axis=-1, keepdims=True)
    z = jnp.sum(jnp.exp(x - m), axis=-1, keepdims=True)
    t = jnp.sum(dec_ref[0] * w_ref[...], axis=-1, keepdims=True) + b_ref[0, 0]
    lpg = jnp.minimum(t, 0.0) - jnp.log(1.0 + jnp.exp(-jnp.abs(t)))
    cc_ref[...] = jnp.broadcast_to(lpg - m - jnp.log(z), (G8, 128))


def _stats(x3, dec3, w2, b2):
    return pl.pallas_call(
        _stats_body,
        grid=(R // G8,),
        in_specs=[
            pl.BlockSpec((1, G8, V), lambda r: (r // 2, r % 2, 0)),
            pl.BlockSpec((1, G8, D), lambda r: (r // 2, r % 2, 0)),
            pl.BlockSpec((1, D), lambda r: (0, 0)),
            pl.BlockSpec((1, 128), lambda r: (0, 0)),
        ],
        out_specs=pl.BlockSpec((G8, 128), lambda r: (r, 0)),
        out_shape=jax.ShapeDtypeStruct((R, 128), jnp.float32),
    )(x3, dec3, w2, b2)


def _mass_body(attn_ref, dec_ref, w_ref, b_ref, enc_ref, encc_ref, s_ref,
               first_ref):
    am = jnp.mean(attn_ref[0], axis=0)
    am = am - jnp.max(am, axis=-1, keepdims=True)
    ea = jnp.exp(am)
    a = ea / jnp.sum(ea, axis=-1, keepdims=True)
    t = jnp.sum(dec_ref[0] * w_ref[...], axis=-1, keepdims=True) + b_ref[0, 0]
    pg = 1.0 / (1.0 + jnp.exp(-t))
    s0 = (1.0 - pg) * a
    eqb = encc_ref[0] == enc_ref[0]
    eq = eqb.astype(jnp.float32)
    s_ref[0] = jnp.dot(s0, eq, preferred_element_type=jnp.float32)
    jj = lax.broadcasted_iota(jnp.int32, (L, L), 0)
    minidx = jnp.min(jnp.where(eqb, jj, L), axis=0, keepdims=True)
    ii = lax.broadcasted_iota(jnp.int32, (1, L), 1)
    first_ref[0] = (minidx == ii).astype(jnp.int32)


def _mass(attn, dec3, w2, b2, enc3, encc3):
    return pl.pallas_call(
        _mass_body,
        grid=(B,),
        in_specs=[
            pl.BlockSpec((1, H, T, L), lambda b: (b, 0, 0, 0)),
            pl.BlockSpec((1, T, D), lambda b: (b, 0, 0)),
            pl.BlockSpec((1, D), lambda b: (0, 0)),
            pl.BlockSpec((1, 128), lambda b: (0, 0)),
            pl.BlockSpec((1, 1, L), lambda b: (b, 0, 0)),
            pl.BlockSpec((1, L, 1), lambda b: (b, 0, 0)),
        ],
        out_specs=[
            pl.BlockSpec((1, T, L), lambda b: (b, 0, 0)),
            pl.BlockSpec((1, 1, L), lambda b: (b, 0, 0)),
        ],
        out_shape=[
            jax.ShapeDtypeStruct((B, T, L), jnp.float32),
            jax.ShapeDtypeStruct((B, 1, L), jnp.int32),
        ],
    )(attn, dec3, w2, b2, enc3, encc3)


@functools.lru_cache(maxsize=None)
def _sc_mesh():
    return plsc.VectorSubcoreMesh(
        core_axis_name="c", subcore_axis_name="s",
        num_cores=NC, num_subcores=NS)


def _nlog(a):
    bits = plsc.bitcast(a, jnp.int32)
    y = bits.astype(jnp.float32) * (_LN2 / (1 << 23)) - (127.0 * _LN2)
    y = y - 1.0 + a * jnp.exp(-y)
    y = y - 1.0 + a * jnp.exp(-y)
    return y


def _win_range(w):
    return w * WMAX, WMAX


def _main_body(x_hbm, cc_hbm, s_hbm, first_hbm, enc_hbm, out_hbm,
               buf0, buf1, tbuf, enc_v, first_v, s_v, cc_v, hcol_v, hidx_v,
               in_sem0, in_sem1, out_sem0, out_sem1):
    cid = lax.axis_index("c")
    sid = lax.axis_index("s")
    g = cid * 8 + sid // 2
    h = sid % 2
    b = g // 2
    t0 = pl.multiple_of((g % 2) * (T // 2), 8)
    r0 = pl.multiple_of(g * G8, 8)

    pltpu.sync_copy(enc_hbm.at[b], enc_v)
    pltpu.sync_copy(first_hbm.at[b, 0], first_v)
    pltpu.sync_copy(cc_hbm.at[pl.ds(r0, G8), pl.ds(0, 128)], cc_v)
    pltpu.sync_copy(s_hbm.at[b, pl.ds(t0, G8), pl.ds(0, L)], s_v)

    bufs = (buf0, buf1)
    in_sems = (in_sem0, in_sem1)
    out_sems = (out_sem0, out_sem1)

    def _compute_window(buf, v0, wlen):
        nvec = wlen // 16

        def shift_one(i, _):
            off = pl.multiple_of(i * 16, 16)
            for r in range(G8):
                ccr = cc_v[r, pl.ds(0, 16)]
                buf[r, pl.ds(off, 16)] = buf[r, pl.ds(off, 16)] + ccr
            return 0
        lax.fori_loop(0, nvec, shift_one, 0, unroll=2)

        def scan_one(j, off):
            jo = pl.multiple_of(j * 16, 16)
            cols = enc_v[pl.ds(jo, 16)]
            m = (cols >= v0) & (cols < v0 + wlen) & (first_v[pl.ds(jo, 16)] > 0)
            idxs = lax.iota(jnp.int32, 16) + jo
            pref = plsc.cumsum(m.astype(jnp.int32))
            pos = off + pref - 1
            plsc.store_scatter(hcol_v, [pos], cols, mask=m)
            plsc.store_scatter(hidx_v, [pos], idxs, mask=m)
            return off + jnp.sum(m.astype(jnp.int32))

        total = lax.fori_loop(0, L // 16, scan_one, 0)
        n_hv = (total + 15) // 16

        def apply_one(kb, _):
            base = kb * 16
            lane_ok = (lax.iota(jnp.int32, 16) + base) < total
            cols = hcol_v[pl.ds(base, 16)] - v0
            sidx = hidx_v[pl.ds(base, 16)]
            cols = jnp.where(lane_ok, cols, 0)
            sidx = jnp.where(lane_ok, sidx, 0)
            for r in range(G8):
                rsplat = jnp.full((16,), r, dtype=jnp.int32)
                vcur = plsc.load_gather(buf, [rsplat, cols], mask=lane_ok)
                sval = plsc.load_gather(s_v, [rsplat, sidx], mask=lane_ok)
                a = jnp.exp(vcur) + sval
                newv = _nlog(a)
                plsc.store_scatter(buf, [rsplat, cols], newv, mask=lane_ok)
            return 0
        lax.fori_loop(0, n_hv, apply_one, 0)

    def _process(wins):
        n = len(wins)
        v0, wlen = _win_range(wins[0])
        pltpu.async_copy(
            x_hbm.at[b, pl.ds(t0, G8), pl.ds(v0, wlen)],
            bufs[0].at[:, pl.ds(0, wlen)], in_sems[0])
        for k in range(n):
            pk = k % 2
            v0, wlen = _win_range(wins[k])
            nvec = wlen // 16
            pltpu.make_async_copy(
                x_hbm.at[b, pl.ds(t0, G8), pl.ds(v0, wlen)],
                bufs[pk].at[:, pl.ds(0, wlen)], in_sems[pk]).wait()
            if k + 1 < n:
                nv0, nwlen = _win_range(wins[k + 1])
                if k >= 1:
                    pv0, pwlen = _win_range(wins[k - 1])
                    pltpu.make_async_copy(
                        bufs[1 - pk].at[:, pl.ds(0, pwlen)],
                        out_hbm.at[b, pl.ds(t0, G8), pl.ds(pv0, pwlen)],
                        out_sems[1 - pk]).wait()
                pltpu.async_copy(
                    x_hbm.at[b, pl.ds(t0, G8), pl.ds(nv0, nwlen)],
                    bufs[1 - pk].at[:, pl.ds(0, nwlen)], in_sems[1 - pk])
            buf = bufs[pk]
            _compute_window(buf, v0, wlen)

            pltpu.async_copy(
                buf.at[:, pl.ds(0, wlen)],
                out_hbm.at[b, pl.ds(t0, G8), pl.ds(v0, wlen)],
                out_sems[pk])
        for k in (n - 2, n - 1):
            if k >= 0:
                pk = k % 2
                v0, wlen = _win_range(wins[k])
                pltpu.make_async_copy(
                    bufs[pk].at[:, pl.ds(0, wlen)],
                    out_hbm.at[b, pl.ds(t0, G8), pl.ds(v0, wlen)],
                    out_sems[pk]).wait()

    @pl.when(h == 0)
    def _():
        _process(H0_WINS)

    @pl.when(h == 1)
    def _():
        _process(H1_WINS)
        pltpu.sync_copy(
            x_hbm.at[b, pl.ds(t0, G8), pl.ds(TAIL_V0, W_TAIL)], tbuf)
        _compute_window(tbuf, TAIL_V0, W_TAIL)
        pltpu.sync_copy(
            tbuf, out_hbm.at[b, pl.ds(t0, G8), pl.ds(TAIL_V0, W_TAIL)])


@functools.lru_cache(maxsize=None)
def _main():
    return pl.kernel(
        _main_body,
        out_type=jax.ShapeDtypeStruct((B, T, V), jnp.float32),
        mesh=_sc_mesh(),
        compiler_params=pltpu.CompilerParams(needs_layout_passes=False),
        scratch_types=[
            pltpu.VMEM((G8, WMAX), jnp.float32),
            pltpu.VMEM((G8, WMAX), jnp.float32),
            pltpu.VMEM((G8, W_TAIL), jnp.float32),
            pltpu.VMEM((L,), jnp.int32),
            pltpu.VMEM((L,), jnp.int32),
            pltpu.VMEM((G8, L), jnp.float32),
            pltpu.VMEM((G8, 128), jnp.float32),
            pltpu.VMEM((L + 16,), jnp.int32),
            pltpu.VMEM((L + 16,), jnp.int32),
            pltpu.SemaphoreType.DMA,
            pltpu.SemaphoreType.DMA,
            pltpu.SemaphoreType.DMA,
            pltpu.SemaphoreType.DMA,
        ],
    )


def kernel(dec_output, final_output, attention_weights, encoder_input,
           inp_shape, tar_shape, batch, training, W, b):
    enc32 = encoder_input.astype(jnp.int32)
    w2 = W.reshape(1, D)
    b2 = jnp.broadcast_to(b.reshape(1, 1), (1, 128))

    cc = _stats(final_output, dec_output, w2, b2)
    s_tot, first = _mass(attention_weights, dec_output, w2, b2,
                         enc32.reshape(B, 1, L), enc32.reshape(B, L, 1))
    return _main()(final_output, cc, s_tot, first, enc32)


# speedup vs baseline: 2.0579x; 2.0579x over previous
import functools

import jax
import jax.numpy as jnp
from jax import lax
from jax.experimental import pallas as pl
from jax.experimental.pallas import tpu as pltpu
from jax.experimental.pallas import tpu_sc as plsc

B, T, L, D, H, V = 8, 16, 512, 1024, 16, 100000
R = B * T
NC, NS = 2, 16
G8 = 8
NG = R // G8
WMAX = 6144
NFULL = 16
W_TAIL = V - NFULL * WMAX
TAIL_V0 = NFULL * WMAX
H0_WINS = list(range(8))
H1_WINS = list(range(8, 16))

_LN2 = 0.6931471805599453

def _stats_body(x_ref, dec_ref, w_ref, b_ref, cc_ref):
    x = x_ref[0]
    m = jnp.max(x, axis=-1, keepdims=True)
    z = jnp.sum(jnp.exp(x - m), axis=-1, keepdims=True)
    t = jnp.sum(dec_ref[0] * w_ref[...], axis=-1, keepdims=True) + b_ref[0, 0]
    lpg = jnp.minimum(t, 0.0) - jnp.log(1.0 + jnp.exp(-jnp.abs(t)))
    cc_ref[...] = jnp.broadcast_to(lpg - m - jnp.log(z), (G8, 128))


def _stats(x3, dec3, w2, b2):
    return pl.pallas_call(
        _stats_body,
        grid=(R // G8,),
        in_specs=[
            pl.BlockSpec((1, G8, V), lambda r: (r // 2, r % 2, 0)),
            pl.BlockSpec((1, G8, D), lambda r: (r // 2, r % 2, 0)),
            pl.BlockSpec((1, D), lambda r: (0, 0)),
            pl.BlockSpec((1, 128), lambda r: (0, 0)),
        ],
        out_specs=pl.BlockSpec((G8, 128), lambda r: (r, 0)),
        out_shape=jax.ShapeDtypeStruct((R, 128), jnp.float32),
    )(x3, dec3, w2, b2)


def _mass_body(attn_ref, dec_ref, w_ref, b_ref, enc_ref, encc_ref, s_ref,
               first_ref):
    am = jnp.mean(attn_ref[0], axis=0)
    am = am - jnp.max(am, axis=-1, keepdims=True)
    ea = jnp.exp(am)
    a = ea / jnp.sum(ea, axis=-1, keepdims=True)
    t = jnp.sum(dec_ref[0] * w_ref[...], axis=-1, keepdims=True) + b_ref[0, 0]
    pg = 1.0 / (1.0 + jnp.exp(-t))
    s0 = (1.0 - pg) * a
    eqb = encc_ref[0] == enc_ref[0]
    eq = eqb.astype(jnp.float32)
    s_ref[0] = jnp.dot(s0, eq, preferred_element_type=jnp.float32)
    jj = lax.broadcasted_iota(jnp.int32, (L, L), 0)
    minidx = jnp.min(jnp.where(eqb, jj, L), axis=0, keepdims=True)
    ii = lax.broadcasted_iota(jnp.int32, (1, L), 1)
    first_ref[0] = (minidx == ii).astype(jnp.int32)


def _mass(attn, dec3, w2, b2, enc3, encc3):
    return pl.pallas_call(
        _mass_body,
        grid=(B,),
        in_specs=[
            pl.BlockSpec((1, H, T, L), lambda b: (b, 0, 0, 0)),
            pl.BlockSpec((1, T, D), lambda b: (b, 0, 0)),
            pl.BlockSpec((1, D), lambda b: (0, 0)),
            pl.BlockSpec((1, 128), lambda b: (0, 0)),
            pl.BlockSpec((1, 1, L), lambda b: (b, 0, 0)),
            pl.BlockSpec((1, L, 1), lambda b: (b, 0, 0)),
        ],
        out_specs=[
            pl.BlockSpec((1, T, L), lambda b: (b, 0, 0)),
            pl.BlockSpec((1, 1, L), lambda b: (b, 0, 0)),
        ],
        out_shape=[
            jax.ShapeDtypeStruct((B, T, L), jnp.float32),
            jax.ShapeDtypeStruct((B, 1, L), jnp.int32),
        ],
    )(attn, dec3, w2, b2, enc3, encc3)


@functools.lru_cache(maxsize=None)
def _sc_mesh():
    return plsc.VectorSubcoreMesh(
        core_axis_name="c", subcore_axis_name="s",
        num_cores=NC, num_subcores=NS)


def _nlog(a):
    bits = plsc.bitcast(a, jnp.int32)
    y = bits.astype(jnp.float32) * (_LN2 / (1 << 23)) - (127.0 * _LN2)
    y = y - 1.0 + a * jnp.exp(-y)
    y = y - 1.0 + a * jnp.exp(-y)
    return y


def _win_range(w):
    return w * WMAX, WMAX


def _main_body(x_hbm, cc_hbm, s_hbm, first_hbm, enc_hbm, out_hbm,
               buf0, buf1, tbuf, enc_v, first_v, s_v, cc_v, hcol_v, hidx_v,
               in_sem0, in_sem1, out_sem0, out_sem1):
    cid = lax.axis_index("c")
    sid = lax.axis_index("s")
    g = cid * 8 + sid // 2
    h = sid % 2
    b = g // 2
    t0 = pl.multiple_of((g % 2) * (T // 2), 8)
    r0 = pl.multiple_of(g * G8, 8)

    pltpu.sync_copy(enc_hbm.at[b], enc_v)
    pltpu.sync_copy(first_hbm.at[b, 0], first_v)
    pltpu.sync_copy(cc_hbm.at[pl.ds(r0, G8), pl.ds(0, 128)], cc_v)
    pltpu.sync_copy(s_hbm.at[b, pl.ds(t0, G8), pl.ds(0, L)], s_v)

    bufs = (buf0, buf1)
    in_sems = (in_sem0, in_sem1)
    out_sems = (out_sem0, out_sem1)

    def _compute_window(buf, v0, wlen):
        nvec = wlen // 16

        @functools.partial(plsc.parallel_loop, 0, nvec, unroll=4)
        def _(i):
            off = pl.multiple_of(i * 16, 16)
            for r in range(G8):
                ccr = cc_v[r, pl.ds(0, 16)]
                buf[r, pl.ds(off, 16)] = buf[r, pl.ds(off, 16)] + ccr

        def scan_one(j, off):
            jo = pl.multiple_of(j * 16, 16)
            cols = enc_v[pl.ds(jo, 16)]
            m = (cols >= v0) & (cols < v0 + wlen) & (first_v[pl.ds(jo, 16)] > 0)
            idxs = lax.iota(jnp.int32, 16) + jo
            pref = plsc.cumsum(m.astype(jnp.int32))
            pos = off + pref - 1
            plsc.store_scatter(hcol_v, [pos], cols, mask=m)
            plsc.store_scatter(hidx_v, [pos], idxs, mask=m)
            return off + jnp.sum(m.astype(jnp.int32))

        total = lax.fori_loop(0, L // 16, scan_one, 0)
        n_hv = (total + 15) // 16

        def apply_one(kb, _):
            base = kb * 16
            lane_ok = (lax.iota(jnp.int32, 16) + base) < total
            cols = hcol_v[pl.ds(base, 16)] - v0
            sidx = hidx_v[pl.ds(base, 16)]
            cols = jnp.where(lane_ok, cols, 0)
            sidx = jnp.where(lane_ok, sidx, 0)
            for r in range(G8):
                rsplat = jnp.full((16,), r, dtype=jnp.int32)
                vcur = plsc.load_gather(buf, [rsplat, cols], mask=lane_ok)
                sval = plsc.load_gather(s_v, [rsplat, sidx], mask=lane_ok)
                a = jnp.exp(vcur) + sval
                newv = _nlog(a)
                plsc.store_scatter(buf, [rsplat, cols], newv, mask=lane_ok)
            return 0
        lax.fori_loop(0, n_hv, apply_one, 0)

    def _process(wins):
        n = len(wins)
        v0, wlen = _win_range(wins[0])
        pltpu.async_copy(
            x_hbm.at[b, pl.ds(t0, G8), pl.ds(v0, wlen)],
            bufs[0].at[:, pl.ds(0, wlen)], in_sems[0])
        for k in range(n):
            pk = k % 2
            v0, wlen = _win_range(wins[k])
            nvec = wlen // 16
            pltpu.make_async_copy(
                x_hbm.at[b, pl.ds(t0, G8), pl.ds(v0, wlen)],
                bufs[pk].at[:, pl.ds(0, wlen)], in_sems[pk]).wait()
            if k + 1 < n:
                nv0, nwlen = _win_range(wins[k + 1])
                if k >= 1:
                    pv0, pwlen = _win_range(wins[k - 1])
                    pltpu.make_async_copy(
                        bufs[1 - pk].at[:, pl.ds(0, pwlen)],
                        out_hbm.at[b, pl.ds(t0, G8), pl.ds(pv0, pwlen)],
                        out_sems[1 - pk]).wait()
                pltpu.async_copy(
                    x_hbm.at[b, pl.ds(t0, G8), pl.ds(nv0, nwlen)],
                    bufs[1 - pk].at[:, pl.ds(0, nwlen)], in_sems[1 - pk])
            buf = bufs[pk]
            _compute_window(buf, v0, wlen)

            pltpu.async_copy(
                buf.at[:, pl.ds(0, wlen)],
                out_hbm.at[b, pl.ds(t0, G8), pl.ds(v0, wlen)],
                out_sems[pk])
        for k in (n - 2, n - 1):
            if k >= 0:
                pk = k % 2
                v0, wlen = _win_range(wins[k])
                pltpu.make_async_copy(
                    bufs[pk].at[:, pl.ds(0, wlen)],
                    out_hbm.at[b, pl.ds(t0, G8), pl.ds(v0, wlen)],
                    out_sems[pk]).wait()

    @pl.when(h == 0)
    def _():
        _process(H0_WINS)

    @pl.when(h == 1)
    def _():
        _process(H1_WINS)
        pltpu.sync_copy(
            x_hbm.at[b, pl.ds(t0, G8), pl.ds(TAIL_V0, W_TAIL)], tbuf)
        _compute_window(tbuf, TAIL_V0, W_TAIL)
        pltpu.sync_copy(
            tbuf, out_hbm.at[b, pl.ds(t0, G8), pl.ds(TAIL_V0, W_TAIL)])


@functools.lru_cache(maxsize=None)
def _main():
    return pl.kernel(
        _main_body,
        out_type=jax.ShapeDtypeStruct((B, T, V), jnp.float32),
        mesh=_sc_mesh(),
        compiler_params=pltpu.CompilerParams(needs_layout_passes=False),
        scratch_types=[
            pltpu.VMEM((G8, WMAX), jnp.float32),
            pltpu.VMEM((G8, WMAX), jnp.float32),
            pltpu.VMEM((G8, W_TAIL), jnp.float32),
            pltpu.VMEM((L,), jnp.int32),
            pltpu.VMEM((L,), jnp.int32),
            pltpu.VMEM((G8, L), jnp.float32),
            pltpu.VMEM((G8, 128), jnp.float32),
            pltpu.VMEM((L + 16,), jnp.int32),
            pltpu.VMEM((L + 16,), jnp.int32),
            pltpu.SemaphoreType.DMA,
            pltpu.SemaphoreType.DMA,
            pltpu.SemaphoreType.DMA,
            pltpu.SemaphoreType.DMA,
        ],
    )


def kernel(dec_output, final_output, attention_weights, encoder_input,
           inp_shape, tar_shape, batch, training, W, b):
    enc32 = encoder_input.astype(jnp.int32)
    w2 = W.reshape(1, D)
    b2 = jnp.broadcast_to(b.reshape(1, 1), (1, 128))

    cc = _stats(final_output, dec_output, w2, b2)
    s_tot, first = _mass(attention_weights, dec_output, w2, b2,
                         enc32.reshape(B, 1, L), enc32.reshape(B, L, 1))
    return _main()(final_output, cc, s_tot, first, enc32)
